# Initial kernel scaffold; baseline (speedup 1.0000x reference)
import jax
import jax.numpy as jnp
from jax.experimental import pallas as pl


def kernel(x, table, W1, b1, W2, b2):
    raise NotImplementedError("write your pallas kernel here")


# speedup vs baseline: 16.5308x; 16.5308x over previous
import functools

import jax
import jax.numpy as jnp
from jax import lax
from jax.experimental import pallas as pl
from jax.experimental.pallas import tpu as pltpu
from jax.experimental.pallas import tpu_sc as plsc

NUM_EMB = 1000000
EMB_DIM = 32
INPUT_LEN = 26
HIDDEN = 256
OUT = 1
B = 16384

N_IDX = B * INPUT_LEN
NUM_WORKERS = 32
N_PER_W = N_IDX // NUM_WORKERS
CHUNK = 1024
N_CHUNKS = N_PER_W // CHUNK

MLP_BLOCK = 512


def _sc_gather(x_flat, table):
    mesh = plsc.VectorSubcoreMesh(core_axis_name="c", subcore_axis_name="s")

    @functools.partial(
        pl.kernel,
        mesh=mesh,
        compiler_params=pltpu.CompilerParams(use_tc_tiling_on_sc=False),
        out_type=jax.ShapeDtypeStruct((N_IDX, EMB_DIM), jnp.float32),
        scratch_types=[
            pltpu.VMEM((CHUNK,), jnp.int32),
            pltpu.VMEM((CHUNK, EMB_DIM), jnp.float32),
            pltpu.SemaphoreType.DMA,
        ],
    )
    def gather_kernel(x_hbm, table_hbm, out_hbm, idx_v, rows_v, sem):
        wid = lax.axis_index("s") * 2 + lax.axis_index("c")
        base = wid * N_PER_W
        for c in range(N_CHUNKS):
            off = base + c * CHUNK
            pltpu.sync_copy(x_hbm.at[pl.ds(off, CHUNK)], idx_v)
            pltpu.async_copy(table_hbm.at[idx_v], rows_v, sem).wait()
            pltpu.sync_copy(rows_v, out_hbm.at[pl.ds(off, CHUNK)])

    return gather_kernel(x_flat, table)


def _mlp_kernel(flat_ref, w1_ref, b1_ref, w2_ref, b2_ref, out_ref):
    h = jnp.dot(flat_ref[...], w1_ref[...], preferred_element_type=jnp.float32)
    h = jnp.maximum(h + b1_ref[...], 0.0)
    out_ref[...] = (
        jnp.dot(h, w2_ref[...], preferred_element_type=jnp.float32) + b2_ref[...]
    )


def _tc_mlp(flat, W1, b1, W2, b2):
    in_dim = INPUT_LEN * EMB_DIM
    grid = (B // MLP_BLOCK,)
    return pl.pallas_call(
        _mlp_kernel,
        grid=grid,
        in_specs=[
            pl.BlockSpec((MLP_BLOCK, in_dim), lambda i: (i, 0)),
            pl.BlockSpec((in_dim, HIDDEN), lambda i: (0, 0)),
            pl.BlockSpec((1, HIDDEN), lambda i: (0, 0)),
            pl.BlockSpec((HIDDEN, OUT), lambda i: (0, 0)),
            pl.BlockSpec((1, OUT), lambda i: (0, 0)),
        ],
        out_specs=pl.BlockSpec((MLP_BLOCK, OUT), lambda i: (i, 0)),
        out_shape=jax.ShapeDtypeStruct((B, OUT), jnp.float32),
    )(flat, W1, b1.reshape(1, HIDDEN), W2, b2.reshape(1, OUT))


def kernel(x, table, W1, b1, W2, b2):
    x_flat = x.reshape(-1)
    rows = _sc_gather(x_flat, table)
    flat = rows.reshape(B, INPUT_LEN * EMB_DIM)
    return _tc_mlp(flat, W1, b1, W2, b2)


# speedup vs baseline: 16.5502x; 1.0012x over previous
import functools

import jax
import jax.numpy as jnp
from jax import lax
from jax.experimental import pallas as pl
from jax.experimental.pallas import tpu as pltpu
from jax.experimental.pallas import tpu_sc as plsc

NUM_EMB = 1000000
EMB_DIM = 32
INPUT_LEN = 26
HIDDEN = 256
OUT = 1
B = 16384
IN_DIM = INPUT_LEN * EMB_DIM

N_IDX = B * INPUT_LEN
NUM_WORKERS = 32
N_PER_W = N_IDX // NUM_WORKERS
CHUNK = 1024
N_CHUNKS = N_PER_W // CHUNK

MLP_BLOCK = 512


def _sc_gather(x_flat, table):
    mesh = plsc.VectorSubcoreMesh(core_axis_name="c", subcore_axis_name="s")

    @functools.partial(
        pl.kernel,
        mesh=mesh,
        compiler_params=pltpu.CompilerParams(use_tc_tiling_on_sc=False),
        out_type=jax.ShapeDtypeStruct((N_IDX, EMB_DIM), jnp.float32),
        scratch_types=[
            pltpu.VMEM((CHUNK,), jnp.int32),
            pltpu.VMEM((CHUNK, EMB_DIM), jnp.float32),
            pltpu.SemaphoreType.DMA,
        ],
    )
    def gather_kernel(x_hbm, table_hbm, out_hbm, idx_v, buf_v, sem):
        wid = lax.axis_index("s") * 2 + lax.axis_index("c")
        base = wid * N_PER_W

        @pl.loop(0, N_CHUNKS)
        def chunk_loop(c):
            off = base + c * CHUNK
            pltpu.sync_copy(x_hbm.at[pl.ds(off, CHUNK)], idx_v)
            pltpu.async_copy(table_hbm.at[idx_v], buf_v, sem).wait()
            pltpu.sync_copy(buf_v, out_hbm.at[pl.ds(off, CHUNK)])

    return gather_kernel(x_flat, table)


def _mlp_kernel(flat_ref, w1_ref, b1_ref, w2_ref, b2_ref, out_ref):
    h = jnp.dot(flat_ref[...], w1_ref[...], preferred_element_type=jnp.float32)
    h = jnp.maximum(h + b1_ref[...], 0.0)
    out_ref[...] = (
        jnp.dot(h, w2_ref[...], preferred_element_type=jnp.float32) + b2_ref[...]
    )


def _tc_mlp(flat, W1, b1, W2, b2):
    grid = (B // MLP_BLOCK,)
    return pl.pallas_call(
        _mlp_kernel,
        grid=grid,
        in_specs=[
            pl.BlockSpec((MLP_BLOCK, IN_DIM), lambda i: (i, 0)),
            pl.BlockSpec((IN_DIM, HIDDEN), lambda i: (0, 0)),
            pl.BlockSpec((1, HIDDEN), lambda i: (0, 0)),
            pl.BlockSpec((HIDDEN, OUT), lambda i: (0, 0)),
            pl.BlockSpec((1, OUT), lambda i: (0, 0)),
        ],
        out_specs=pl.BlockSpec((MLP_BLOCK, OUT), lambda i: (i, 0)),
        out_shape=jax.ShapeDtypeStruct((B, OUT), jnp.float32),
    )(flat, W1, b1.reshape(1, HIDDEN), W2, b2.reshape(1, OUT))


def kernel(x, table, W1, b1, W2, b2):
    table_lin = lax.optimization_barrier(table.reshape(-1))
    table_rm = table_lin.reshape(NUM_EMB, EMB_DIM)
    rows = _sc_gather(x.reshape(-1), table_rm)
    flat = rows.reshape(B, IN_DIM)
    return _tc_mlp(flat, W1, b1, W2, b2)
